# Initial kernel scaffold; baseline (speedup 1.0000x reference)
import jax
import jax.numpy as jnp
from jax.experimental import pallas as pl


def kernel(x, codebook):
    raise NotImplementedError("write your pallas kernel here")


# speedup vs baseline: 5.1886x; 5.1886x over previous
import jax
import jax.numpy as jnp
from jax import lax
from jax.experimental import pallas as pl

_K = 512
_DG = 32
_G = 2


def _vq_tc_body(xg_ref, cb_ref, q_ref, idx_ref):
    xg = xg_ref[0]
    cb = cb_ref[...]
    T = xg.shape[1]
    dots = lax.dot_general(cb, xg, (((1,), (0,)), ((), ())),
                           precision=lax.Precision.HIGHEST,
                           preferred_element_type=jnp.float32)
    cn = jnp.sum(cb * cb, axis=1, keepdims=True)
    scores = cn - 2.0 * dots
    m = jnp.min(scores, axis=0, keepdims=True)
    kiota = lax.broadcasted_iota(jnp.int32, (_K, T), 0)
    masked = jnp.where(scores == m, kiota, _K)
    idx_row = jnp.min(masked, axis=0, keepdims=True)
    oh = (kiota == idx_row).astype(jnp.float32)
    q = lax.dot_general(cb, oh, (((0,), (0,)), ((), ())),
                        precision=lax.Precision.HIGHEST,
                        preferred_element_type=jnp.float32)
    q_ref[0] = q
    idx_ref[0] = idx_row


def kernel(x, codebook):
    B, C, T = x.shape
    xg = x.reshape(B, _DG, _G, T).transpose(2, 0, 1, 3).reshape(_G * B, _DG, T)
    q, idx = pl.pallas_call(
        _vq_tc_body,
        grid=(_G * B,),
        in_specs=[
            pl.BlockSpec((1, _DG, T), lambda i: (i, 0, 0)),
            pl.BlockSpec((_K, _DG), lambda i: (0, 0)),
        ],
        out_specs=[
            pl.BlockSpec((1, _DG, T), lambda i: (i % B, i // B, 0)),
            pl.BlockSpec((1, 1, T), lambda i: (i, 0, 0)),
        ],
        out_shape=[
            jax.ShapeDtypeStruct((B, C, T), jnp.float32),
            jax.ShapeDtypeStruct((_G * B, 1, T), jnp.int32),
        ],
    )(xg, codebook)
    return q, idx.reshape(_G, B, T)
